# baseline (speedup 1.0000x reference)
import functools

import jax
import jax.numpy as jnp
from jax import lax
from jax.experimental import pallas as pl
from jax.experimental.pallas import tpu as pltpu
from jax.experimental.pallas import tpu_sc as plsc

N_GRAPHS = 1024
EMB_ATOMIC = 64
N_CHARGE = 32
EMB_CHARGE = 16
CONT_IN = 8
EMB_CONT = 32
CAT_DIM = EMB_ATOMIC + EMB_CHARGE
TOTAL_DIM = CAT_DIM + EMB_CONT
OUT_DIM = 128

NW = 32
SL = 128
STRIPS = 25
SS_DMAS = 5
SS = SL * SS_DMAS
PER_W = SL * STRIPS
NPAD = NW * PER_W
CHUNKS = ((0, 20), (640, 5))
BLK = 4096


def _prep_body(tc_ref, ec_ref, gg_ref):
    tc = tc_ref[...]
    oh = (tc == lax.broadcasted_iota(jnp.int32, (N_GRAPHS, N_CHARGE), 1))
    gg_ref[...] = jnp.dot(oh.astype(jnp.float32), ec_ref[...],
                          preferred_element_type=jnp.float32)


_prep = pl.pallas_call(
    _prep_body,
    out_shape=jax.ShapeDtypeStruct((N_GRAPHS, EMB_CHARGE), jnp.float32),
)


@functools.lru_cache(maxsize=None)
def _make_gather(row_base, spw):
  supers = [min(SS_DMAS, spw - s) for s in range(0, spw, SS_DMAS)]
  nsup = len(supers)

  def body(at_hbm, b_hbm, table_hbm, gg_hbm, x_out,
           at_v, b_v, a0, a1, c0, c1, sem0, sem1, osem0, osem1):
    cid = lax.axis_index("c")
    sid = lax.axis_index("s")
    wid = cid * 16 + sid
    idx_row0 = row_base + wid * spw
    node_base = wid * spw * SL
    pltpu.sync_copy(at_hbm.at[pl.ds(idx_row0, spw)], at_v)
    pltpu.sync_copy(b_hbm.at[pl.ds(idx_row0, spw)], b_v)
    abufs, cbufs = (a0, a1), (c0, c1)
    sems, osems = (sem0, sem1), (osem0, osem1)
    pending = {}
    draining = {}

    def start(j):
        p = j & 1
        ds = []
        for k in range(supers[j]):
            r = j * SS_DMAS + k
            ds.append(pltpu.async_copy(
                table_hbm.at[at_v.at[r]], abufs[p].at[pl.ds(k * SL, SL)],
                sems[p]))
            ds.append(pltpu.async_copy(
                gg_hbm.at[b_v.at[r]], cbufs[p].at[pl.ds(k * SL, SL)],
                sems[p]))
        pending[j] = ds

    def drain(j):
        p = j & 1
        m = supers[j] * SL
        rows = pl.ds(node_base + j * SS, m)
        draining[j] = (
            pltpu.async_copy(abufs[p].at[pl.ds(0, m)],
                             x_out.at[rows, pl.ds(0, EMB_ATOMIC)],
                             osems[p]),
            pltpu.async_copy(cbufs[p].at[pl.ds(0, m)],
                             x_out.at[rows, pl.ds(EMB_ATOMIC, EMB_CHARGE)],
                             osems[p]),
        )

    start(0)
    for j in range(nsup):
        if j >= 1:
            for d in draining.pop(j - 1):
                d.wait()
        if j + 1 < nsup:
            start(j + 1)
        for d in pending.pop(j):
            d.wait()
        drain(j)
    for d in draining.pop(nsup - 1):
        d.wait()

  return pl.kernel(
    body,
    out_type=jax.ShapeDtypeStruct((NW * spw * SL, OUT_DIM), jnp.float32),
    mesh=plsc.VectorSubcoreMesh(core_axis_name="c", subcore_axis_name="s",
                                num_cores=2, num_subcores=16),
    compiler_params=pltpu.CompilerParams(use_tc_tiling_on_sc=False),
    scratch_types=[
        pltpu.VMEM((spw, SL), jnp.int32),
        pltpu.VMEM((spw, SL), jnp.int32),
        pltpu.VMEM((SS, EMB_ATOMIC), jnp.float32),
        pltpu.VMEM((SS, EMB_ATOMIC), jnp.float32),
        pltpu.VMEM((SS, EMB_CHARGE), jnp.float32),
        pltpu.VMEM((SS, EMB_CHARGE), jnp.float32),
        pltpu.SemaphoreType.DMA,
        pltpu.SemaphoreType.DMA,
        pltpu.SemaphoreType.DMA,
        pltpu.SemaphoreType.DMA,
    ],
  )


def _dense_body(x_ref, ef_ref, wp_ref, w1_ref, b1_ref, w2_ref, b2_ref,
                out_ref):
    h = jnp.dot(ef_ref[...], w1_ref[...],
                preferred_element_type=jnp.float32) + b1_ref[...]
    h = h * lax.logistic(h)
    wp = wp_ref[...]
    wh = jnp.dot(w2_ref[...], wp[CAT_DIM:], preferred_element_type=jnp.float32)
    bias = jnp.dot(b2_ref[...], wp[CAT_DIM:],
                   preferred_element_type=jnp.float32)
    y = jnp.dot(x_ref[...][:, :CAT_DIM], wp[:CAT_DIM],
                preferred_element_type=jnp.float32)
    y = y + jnp.dot(h, wh, preferred_element_type=jnp.float32) + bias
    out_ref[...] = y * lax.logistic(y)


def _dense_body_aliased(y_prev_ref, x_ref, ef_ref, wp_ref, w1_ref, b1_ref,
                        w2_ref, b2_ref, out_ref):
    del y_prev_ref
    _dense_body(x_ref, ef_ref, wp_ref, w1_ref, b1_ref, w2_ref, b2_ref,
                out_ref)


@functools.lru_cache(maxsize=None)
def _make_dense(n, blk0, nblk, aliased):
    specs = [
        pl.BlockSpec((BLK, OUT_DIM), lambda b: (b, 0)),
        pl.BlockSpec((BLK, CONT_IN), lambda b: (blk0 + b, 0)),
        pl.BlockSpec((TOTAL_DIM, OUT_DIM), lambda b: (0, 0)),
        pl.BlockSpec((CONT_IN, EMB_CONT), lambda b: (0, 0)),
        pl.BlockSpec((1, EMB_CONT), lambda b: (0, 0)),
        pl.BlockSpec((EMB_CONT, EMB_CONT), lambda b: (0, 0)),
        pl.BlockSpec((1, EMB_CONT), lambda b: (0, 0)),
    ]
    body = _dense_body
    kwargs = {}
    if aliased:
        specs = [pl.BlockSpec(memory_space=pl.ANY)] + specs
        body = _dense_body_aliased
        kwargs["input_output_aliases"] = {0: 0}
    return pl.pallas_call(
        body,
        grid=(nblk,),
        in_specs=specs,
        out_specs=pl.BlockSpec((BLK, OUT_DIM), lambda b: (blk0 + b, 0)),
        out_shape=jax.ShapeDtypeStruct((n, OUT_DIM), jnp.float32),
        **kwargs,
    )


def kernel(batch, atomic_type, total_charge, external_field,
           emb_atomic, emb_charge, W1, b1, W2, b2, Wp):
    n = batch.shape[0]
    pad = NPAD - n
    at = jnp.pad(atomic_type.astype(jnp.int32).reshape(-1), (0, pad))
    bt = jnp.pad(batch.astype(jnp.int32), (0, pad))
    at2 = at.reshape(NW * STRIPS, SL)
    bt2 = bt.reshape(NW * STRIPS, SL)

    gg = _prep(total_charge.astype(jnp.int32).reshape(-1, 1), emb_charge)
    ws = (Wp, W1, b1.reshape(1, -1), W2, b2.reshape(1, -1))

    y = None
    for row_base, spw in CHUNKS:
        x_c = _make_gather(row_base, spw)(at2, bt2, emb_atomic, gg)
        blk0 = row_base * SL // BLK
        nblk = min(-(-n // BLK) - blk0, NW * spw * SL // BLK)
        dense = _make_dense(n, blk0, nblk, y is not None)
        args = (x_c, external_field) + ws
        y = dense(y, *args) if y is not None else dense(*args)
    return y


# speedup vs baseline: 1.0189x; 1.0189x over previous
import functools

import jax
import jax.numpy as jnp
from jax import lax
from jax.experimental import pallas as pl
from jax.experimental.pallas import tpu as pltpu
from jax.experimental.pallas import tpu_sc as plsc

N_GRAPHS = 1024
EMB_ATOMIC = 64
N_CHARGE = 32
EMB_CHARGE = 16
CONT_IN = 8
EMB_CONT = 32
CAT_DIM = EMB_ATOMIC + EMB_CHARGE
TOTAL_DIM = CAT_DIM + EMB_CONT
OUT_DIM = 128

NW = 32
SL = 128
STRIPS = 25
SS_DMAS = 5
SS = SL * SS_DMAS
PER_W = SL * STRIPS
NPAD = NW * PER_W
CHUNKS = ((0, 16), (512, 9))
BLK = 4096


def _prep_body(tc_ref, ec_ref, gg_ref):
    tc = tc_ref[...]
    oh = (tc == lax.broadcasted_iota(jnp.int32, (N_GRAPHS, N_CHARGE), 1))
    gg_ref[...] = jnp.dot(oh.astype(jnp.float32), ec_ref[...],
                          preferred_element_type=jnp.float32)


_prep = pl.pallas_call(
    _prep_body,
    out_shape=jax.ShapeDtypeStruct((N_GRAPHS, EMB_CHARGE), jnp.float32),
)


@functools.lru_cache(maxsize=None)
def _make_gather(row_base, spw):
  supers = [min(SS_DMAS, spw - s) for s in range(0, spw, SS_DMAS)]
  nsup = len(supers)

  def body(at_hbm, b_hbm, table_hbm, gg_hbm, x_out,
           at_v, b_v, a0, a1, c0, c1, sem0, sem1, osem0, osem1):
    cid = lax.axis_index("c")
    sid = lax.axis_index("s")
    wid = cid * 16 + sid
    idx_row0 = row_base + wid * spw
    node_base = wid * spw * SL
    pltpu.sync_copy(at_hbm.at[pl.ds(idx_row0, spw)], at_v)
    pltpu.sync_copy(b_hbm.at[pl.ds(idx_row0, spw)], b_v)
    abufs, cbufs = (a0, a1), (c0, c1)
    sems, osems = (sem0, sem1), (osem0, osem1)
    pending = {}
    draining = {}

    def start(j):
        p = j & 1
        ds = []
        for k in range(supers[j]):
            r = j * SS_DMAS + k
            ds.append(pltpu.async_copy(
                table_hbm.at[at_v.at[r]], abufs[p].at[pl.ds(k * SL, SL)],
                sems[p]))
            ds.append(pltpu.async_copy(
                gg_hbm.at[b_v.at[r]], cbufs[p].at[pl.ds(k * SL, SL)],
                sems[p]))
        pending[j] = ds

    def drain(j):
        p = j & 1
        m = supers[j] * SL
        rows = pl.ds(node_base + j * SS, m)
        draining[j] = (
            pltpu.async_copy(abufs[p].at[pl.ds(0, m)],
                             x_out.at[rows, pl.ds(0, EMB_ATOMIC)],
                             osems[p]),
            pltpu.async_copy(cbufs[p].at[pl.ds(0, m)],
                             x_out.at[rows, pl.ds(EMB_ATOMIC, EMB_CHARGE)],
                             osems[p]),
        )

    start(0)
    for j in range(nsup):
        if j >= 1:
            for d in draining.pop(j - 1):
                d.wait()
        if j + 1 < nsup:
            start(j + 1)
        for d in pending.pop(j):
            d.wait()
        drain(j)
    for d in draining.pop(nsup - 1):
        d.wait()

  return pl.kernel(
    body,
    out_type=jax.ShapeDtypeStruct((NW * spw * SL, OUT_DIM), jnp.float32),
    mesh=plsc.VectorSubcoreMesh(core_axis_name="c", subcore_axis_name="s",
                                num_cores=2, num_subcores=16),
    compiler_params=pltpu.CompilerParams(use_tc_tiling_on_sc=False),
    scratch_types=[
        pltpu.VMEM((spw, SL), jnp.int32),
        pltpu.VMEM((spw, SL), jnp.int32),
        pltpu.VMEM((SS, EMB_ATOMIC), jnp.float32),
        pltpu.VMEM((SS, EMB_ATOMIC), jnp.float32),
        pltpu.VMEM((SS, EMB_CHARGE), jnp.float32),
        pltpu.VMEM((SS, EMB_CHARGE), jnp.float32),
        pltpu.SemaphoreType.DMA,
        pltpu.SemaphoreType.DMA,
        pltpu.SemaphoreType.DMA,
        pltpu.SemaphoreType.DMA,
    ],
  )


def _dense_body(x_ref, ef_ref, wp_ref, w1_ref, b1_ref, w2_ref, b2_ref,
                out_ref):
    h = jnp.dot(ef_ref[...], w1_ref[...],
                preferred_element_type=jnp.float32) + b1_ref[...]
    h = h * lax.logistic(h)
    wp = wp_ref[...]
    wh = jnp.dot(w2_ref[...], wp[CAT_DIM:], preferred_element_type=jnp.float32)
    bias = jnp.dot(b2_ref[...], wp[CAT_DIM:],
                   preferred_element_type=jnp.float32)
    y = jnp.dot(x_ref[...][:, :CAT_DIM], wp[:CAT_DIM],
                preferred_element_type=jnp.float32)
    y = y + jnp.dot(h, wh, preferred_element_type=jnp.float32) + bias
    out_ref[...] = y * lax.logistic(y)


def _dense_body_aliased(y_prev_ref, x_ref, ef_ref, wp_ref, w1_ref, b1_ref,
                        w2_ref, b2_ref, out_ref):
    del y_prev_ref
    _dense_body(x_ref, ef_ref, wp_ref, w1_ref, b1_ref, w2_ref, b2_ref,
                out_ref)


@functools.lru_cache(maxsize=None)
def _make_dense(n, blk0, nblk, aliased):
    specs = [
        pl.BlockSpec((BLK, OUT_DIM), lambda b: (b, 0)),
        pl.BlockSpec((BLK, CONT_IN), lambda b: (blk0 + b, 0)),
        pl.BlockSpec((TOTAL_DIM, OUT_DIM), lambda b: (0, 0)),
        pl.BlockSpec((CONT_IN, EMB_CONT), lambda b: (0, 0)),
        pl.BlockSpec((1, EMB_CONT), lambda b: (0, 0)),
        pl.BlockSpec((EMB_CONT, EMB_CONT), lambda b: (0, 0)),
        pl.BlockSpec((1, EMB_CONT), lambda b: (0, 0)),
    ]
    body = _dense_body
    kwargs = {}
    if aliased:
        specs = [pl.BlockSpec(memory_space=pl.ANY)] + specs
        body = _dense_body_aliased
        kwargs["input_output_aliases"] = {0: 0}
    return pl.pallas_call(
        body,
        grid=(nblk,),
        in_specs=specs,
        out_specs=pl.BlockSpec((BLK, OUT_DIM), lambda b: (blk0 + b, 0)),
        out_shape=jax.ShapeDtypeStruct((n, OUT_DIM), jnp.float32),
        **kwargs,
    )


def kernel(batch, atomic_type, total_charge, external_field,
           emb_atomic, emb_charge, W1, b1, W2, b2, Wp):
    n = batch.shape[0]
    pad = NPAD - n
    at = jnp.pad(atomic_type.astype(jnp.int32).reshape(-1), (0, pad))
    bt = jnp.pad(batch.astype(jnp.int32), (0, pad))
    at2 = at.reshape(NW * STRIPS, SL)
    bt2 = bt.reshape(NW * STRIPS, SL)

    gg = _prep(total_charge.astype(jnp.int32).reshape(-1, 1), emb_charge)
    ws = (Wp, W1, b1.reshape(1, -1), W2, b2.reshape(1, -1))

    y = None
    for row_base, spw in CHUNKS:
        x_c = _make_gather(row_base, spw)(at2, bt2, emb_atomic, gg)
        blk0 = row_base * SL // BLK
        nblk = min(-(-n // BLK) - blk0, NW * spw * SL // BLK)
        dense = _make_dense(n, blk0, nblk, y is not None)
        args = (x_c, external_field) + ws
        y = dense(y, *args) if y is not None else dense(*args)
    return y
